# baseline (speedup 1.0000x reference)
import functools

import jax
import jax.numpy as jnp
from jax import lax
from jax.experimental import pallas as pl
from jax.experimental.pallas import tpu as pltpu
from jax.experimental.pallas import tpu_sc as plsc

N = 10000
E = 320000
DN = 128
DE = 16
NC = 2
NS = 16
DH = DN // NC
CHUNK = 128
NCHUNK = 160
NBUF = 2
EPT = NCHUNK * CHUNK
EPAD = NS * EPT
NP = 10240
RPT = NP // NS
WB = 320


def _sc_segment_sums(x_flat, row4, col3, attr3, ones_blk, z64, z16):
    mesh = plsc.VectorSubcoreMesh(core_axis_name="c", subcore_axis_name="s")

    @functools.partial(
        pl.kernel,
        mesh=mesh,
        compiler_params=pltpu.CompilerParams(use_tc_tiling_on_sc=False),
        out_type=[
            jax.ShapeDtypeStruct((NC, NP, DH), jnp.float32),
            jax.ShapeDtypeStruct((NC, NP, DE), jnp.float32),
        ],
        scratch_types=[
            pltpu.VMEM((NCHUNK, CHUNK), jnp.int32),
        ] + [pltpu.VMEM((CHUNK,), jnp.int32)] * NBUF
          + [pltpu.VMEM((CHUNK, DH), jnp.float32)] * NBUF
          + [pltpu.VMEM((CHUNK, DE), jnp.float32)] * NBUF
          + [
            pltpu.VMEM((CHUNK, DE), jnp.float32),
            pltpu.VMEM((WB, DH), jnp.float32),
            pltpu.VMEM((WB, DE), jnp.float32),
            pltpu.VMEM_SHARED((NP, DH), jnp.float32),
            pltpu.VMEM_SHARED((NP, DE), jnp.float32),
        ] + [pltpu.SemaphoreType.DMA] * (5 * NBUF),
    )
    def k(x_hbm, row_hbm, col_hbm, attr_hbm, ones_hbm, z64_hbm, z16_hbm,
          sx_out, se_out,
          cidx, *bufs_and_sems):
        ridx = list(bufs_and_sems[0:NBUF])
        rows = list(bufs_and_sems[NBUF:2 * NBUF])
        attr = list(bufs_and_sems[2 * NBUF:3 * NBUF])
        onesb, stx, ste, sx_acc, se_acc = bufs_and_sems[3 * NBUF:3 * NBUF + 5]
        sems = bufs_and_sems[3 * NBUF + 5:]
        isem = sems[0 * NBUF:1 * NBUF]
        gsem = sems[1 * NBUF:2 * NBUF]
        ssem = sems[2 * NBUF:3 * NBUF]
        asem = sems[3 * NBUF:4 * NBUF]
        tsem = sems[4 * NBUF:5 * NBUF]
        c = lax.axis_index("c")
        s = lax.axis_index("s")
        rbase = s * RPT
        on_c0 = c == 0

        pltpu.sync_copy(z64_hbm, stx)
        pltpu.sync_copy(z16_hbm, ste)
        for w in range(RPT // WB):
            pltpu.sync_copy(stx, sx_acc.at[pl.ds(rbase + w * WB, WB)])
            pltpu.sync_copy(ste, se_acc.at[pl.ds(rbase + w * WB, WB)])

        pltpu.sync_copy(col_hbm.at[s], cidx)
        pltpu.sync_copy(ones_hbm, onesb)
        plsc.subcore_barrier()

        ablk = s * NCHUNK

        def drain(j, jn, b):
            pltpu.make_async_copy(x_hbm.at[pl.ds(0, CHUNK)],
                                  rows[b], gsem[b]).wait()
            pltpu.async_copy(row_hbm.at[c, s, jn], ridx[b], isem[b])
            pltpu.async_copy(rows[b], sx_acc.at[cidx.at[j]], ssem[b],
                             add=True)

            @pl.when(on_c0)
            def _():
                pltpu.make_async_copy(attr_hbm.at[0], attr[b],
                                      asem[b]).wait()
                pltpu.async_copy(attr[b], se_acc.at[cidx.at[j]], tsem[b],
                                 add=True)

            @pl.when(jnp.logical_not(on_c0))
            def _():
                pltpu.async_copy(onesb, se_acc.at[cidx.at[j]], tsem[b],
                                 add=True)

        def issue(j, b):
            pltpu.make_async_copy(x_hbm.at[pl.ds(0, CHUNK)],
                                  rows[b], ssem[b]).wait()
            pltpu.make_async_copy(row_hbm.at[c, s, 0], ridx[b],
                                  isem[b]).wait()
            pltpu.async_copy(x_hbm.at[ridx[b]], rows[b], gsem[b])

            @pl.when(on_c0)
            def _():
                pltpu.make_async_copy(attr_hbm.at[0], attr[b],
                                      tsem[b]).wait()
                pltpu.async_copy(attr_hbm.at[ablk + j], attr[b], asem[b])

            @pl.when(jnp.logical_not(on_c0))
            def _():
                pltpu.make_async_copy(attr_hbm.at[0], attr[b],
                                      tsem[b]).wait()

        for b in range(NBUF):
            pltpu.async_copy(row_hbm.at[c, s, b], ridx[b], isem[b])
        for b in range(NBUF):
            pltpu.make_async_copy(row_hbm.at[c, s, 0], ridx[b],
                                  isem[b]).wait()
            pltpu.async_copy(x_hbm.at[ridx[b]], rows[b], gsem[b])

            @pl.when(on_c0)
            def _():
                pltpu.async_copy(attr_hbm.at[ablk + b], attr[b], asem[b])

        def body(i, carry):
            j0 = NBUF * i
            for b in range(NBUF):
                drain(j0 + b, j0 + b + NBUF, b)
            for b in range(NBUF):
                issue(j0 + b + NBUF, b)
            return carry

        lax.fori_loop(0, NCHUNK // NBUF - 1, body, 0)
        for b in range(NBUF):
            drain(NCHUNK - NBUF + b, 0, b)
        for b in range(NBUF):
            pltpu.make_async_copy(row_hbm.at[c, s, 0], ridx[b],
                                  isem[b]).wait()
            pltpu.make_async_copy(x_hbm.at[pl.ds(0, CHUNK)],
                                  rows[b], ssem[b]).wait()
            pltpu.make_async_copy(attr_hbm.at[0], attr[b], tsem[b]).wait()
        plsc.subcore_barrier()

        for w in range(RPT // WB):
            pltpu.sync_copy(sx_acc.at[pl.ds(rbase + w * WB, WB)], stx)
            pltpu.sync_copy(stx, sx_out.at[c, pl.ds(rbase + w * WB, WB)])
            pltpu.sync_copy(se_acc.at[pl.ds(rbase + w * WB, WB)], ste)
            pltpu.sync_copy(ste, se_out.at[c, pl.ds(rbase + w * WB, WB)])

    return k(x_flat, row4, col3, attr3, ones_blk, z64, z16)


def _tc_epilogue(x, sx, se,
                 W1_0, b1_0, W1_1, b1_1, W1_2, b1_2,
                 W2_0, b2_0, W2_1, b2_1, W2_2, b2_2):
    def body(x_ref, sx_ref, se_ref,
             w10, b10, w11, b11, w12, b12,
             w20, b20, w21, b21, w22, b22, out_ref):
        f32 = jnp.float32
        dot = lambda a, b: jnp.dot(a, b, preferred_element_type=f32)
        t12 = dot(w11[...], w12[...])
        w1c = dot(w10[...], t12)
        b1c = dot(b10[...], t12) + dot(b11[...], w12[...]) + b12[...]
        t22 = dot(w21[...], w22[...])
        w2c = dot(w20[...], t22)
        b2c = dot(b20[...], t22) + dot(b21[...], w22[...]) + b22[...]
        p = w2c[:DN]
        bmat = w2c[DN:]
        q = dot(w1c[:DN], bmat)
        r = dot(w1c[DN:], bmat)
        svec = dot(b1c, bmat)
        out_ref[...] = (dot(x_ref[...], p)
                        + dot(sx_ref[0, :N], q[:DH])
                        + dot(sx_ref[1, :N], q[DH:])
                        + dot(se_ref[0, :N], r)
                        + se_ref[1, :N, 0:1] * svec + b2c)

    return pl.pallas_call(
        body,
        out_shape=jax.ShapeDtypeStruct((N, DN), jnp.float32),
    )(x, sx, se,
      W1_0, b1_0.reshape(1, -1), W1_1, b1_1.reshape(1, -1),
      W1_2, b1_2.reshape(1, -1),
      W2_0, b2_0.reshape(1, -1), W2_1, b2_1.reshape(1, -1),
      W2_2, b2_2.reshape(1, -1))


def kernel(x, edge_index, edge_attr, u, batch,
           W1_0, b1_0, W1_1, b1_1, W1_2, b1_2,
           W2_0, b2_0, W2_1, b2_1, W2_2, b2_2):
    pad = EPAD - E
    row = jnp.concatenate([edge_index[0], jnp.zeros((pad,), jnp.int32)])
    col = jnp.concatenate([edge_index[1], jnp.full((pad,), N, jnp.int32)])
    row4 = jnp.stack([row, row + N]).reshape(NC, NS, NCHUNK, CHUNK)
    col3 = col.reshape(NS, NCHUNK, CHUNK)
    x_flat = jnp.concatenate([x[:, :DH], x[:, DH:]])
    attr3 = jnp.concatenate(
        [edge_attr, jnp.zeros((pad, DE), jnp.float32)]
    ).reshape(NS * NCHUNK, CHUNK, DE)
    ones_blk = jnp.ones((CHUNK, DE), jnp.float32)
    z64 = jnp.zeros((WB, DH), jnp.float32)
    z16 = jnp.zeros((WB, DE), jnp.float32)
    sx, se = _sc_segment_sums(x_flat, row4, col3, attr3, ones_blk, z64, z16)
    return _tc_epilogue(x, sx, se,
                        W1_0, b1_0, W1_1, b1_1, W1_2, b1_2,
                        W2_0, b2_0, W2_1, b2_1, W2_2, b2_2)


# speedup vs baseline: 1.1502x; 1.1502x over previous
import functools

import jax
import jax.numpy as jnp
from jax import lax
from jax.experimental import pallas as pl
from jax.experimental.pallas import tpu as pltpu
from jax.experimental.pallas import tpu_sc as plsc

N = 10000
E = 320000
DN = 128
DE = 16
NC = 2
NS = 16
DH = DN // NC
CHUNK = 128
NCHUNK = 158
EPT = NCHUNK * CHUNK
EPAD = NS * EPT
NP = 10240
RPT = NP // NS
WB = 320
NV = CHUNK // 16


def _sc_segment_sums(x2, row3, col3, attr3, ones_blk, z64, z16):
    mesh = plsc.VectorSubcoreMesh(core_axis_name="c", subcore_axis_name="s")

    @functools.partial(
        pl.kernel,
        mesh=mesh,
        compiler_params=pltpu.CompilerParams(use_tc_tiling_on_sc=False),
        out_type=[
            jax.ShapeDtypeStruct((NC, NP, DH), jnp.float32),
            jax.ShapeDtypeStruct((NC, NP, DE), jnp.float32),
        ],
        scratch_types=[
            pltpu.VMEM((NCHUNK, CHUNK), jnp.int32),
            pltpu.VMEM((CHUNK,), jnp.int32),
            pltpu.VMEM((CHUNK,), jnp.int32),
            pltpu.VMEM((CHUNK, DH), jnp.float32),
            pltpu.VMEM((CHUNK, DH), jnp.float32),
            pltpu.VMEM((CHUNK, DE), jnp.float32),
            pltpu.VMEM((CHUNK, DE), jnp.float32),
            pltpu.VMEM((CHUNK, DE), jnp.float32),
            pltpu.VMEM((WB, DH), jnp.float32),
            pltpu.VMEM((WB, DE), jnp.float32),
            pltpu.VMEM_SHARED((NP, DH), jnp.float32),
            pltpu.VMEM_SHARED((NP, DE), jnp.float32),
            pltpu.SemaphoreType.DMA,
            pltpu.SemaphoreType.DMA,
            pltpu.SemaphoreType.DMA,
            pltpu.SemaphoreType.DMA,
            pltpu.SemaphoreType.DMA,
            pltpu.SemaphoreType.DMA,
        ],
    )
    def k(x_hbm, row_hbm, col_hbm, attr_hbm, ones_hbm, z64_hbm, z16_hbm,
          sx_out, se_out,
          cidx, ridx0, ridx1, rows0, rows1, attr0, attr1, onesb, stx, ste,
          sx_acc, se_acc, i0, i1, g0, g1, a0, a1):
        c = lax.axis_index("c")
        s = lax.axis_index("s")
        rbase = s * RPT
        on_c0 = c == 0

        pltpu.sync_copy(z64_hbm, stx)
        pltpu.sync_copy(z16_hbm, ste)
        for w in range(RPT // WB):
            pltpu.sync_copy(stx, sx_acc.at[pl.ds(rbase + w * WB, WB)])
            pltpu.sync_copy(ste, se_acc.at[pl.ds(rbase + w * WB, WB)])

        pltpu.sync_copy(col_hbm.at[s], cidx)
        pltpu.sync_copy(ones_hbm, onesb)
        plsc.subcore_barrier()

        ablk = s * NCHUNK

        def issue(j, ibuf, rbuf, abuf, isem, gsem, asem):
            pltpu.make_async_copy(row_hbm.at[s, 0], ibuf, isem).wait()
            for v in range(NV):
                ibuf[pl.ds(v * 16, 16)] = ibuf[pl.ds(v * 16, 16)] + c
            pltpu.async_copy(x_hbm.at[ibuf], rbuf, gsem)

            @pl.when(on_c0)
            def _():
                pltpu.async_copy(attr_hbm.at[ablk + j], abuf, asem)

        def drain(j, jn, ibuf, rbuf, abuf, isem, gsem, asem):
            pltpu.make_async_copy(x_hbm.at[pl.ds(0, CHUNK)], rbuf, gsem).wait()
            pltpu.async_copy(row_hbm.at[s, jn], ibuf, isem)
            pltpu.sync_copy(rbuf, sx_acc.at[cidx.at[j]], add=True)

            @pl.when(on_c0)
            def _():
                pltpu.make_async_copy(attr_hbm.at[0], abuf, asem).wait()
                pltpu.sync_copy(abuf, se_acc.at[cidx.at[j]], add=True)

            @pl.when(jnp.logical_not(on_c0))
            def _():
                pltpu.sync_copy(onesb, se_acc.at[cidx.at[j]], add=True)

        pltpu.async_copy(row_hbm.at[s, 0], ridx0, i0)
        pltpu.async_copy(row_hbm.at[s, 1], ridx1, i1)
        issue(0, ridx0, rows0, attr0, i0, g0, a0)
        issue(1, ridx1, rows1, attr1, i1, g1, a1)

        def body(i, carry):
            j0 = 2 * i
            drain(j0, j0 + 2, ridx0, rows0, attr0, i0, g0, a0)
            issue(j0 + 2, ridx0, rows0, attr0, i0, g0, a0)
            drain(j0 + 1, j0 + 3, ridx1, rows1, attr1, i1, g1, a1)
            issue(j0 + 3, ridx1, rows1, attr1, i1, g1, a1)
            return carry

        lax.fori_loop(0, NCHUNK // 2 - 1, body, 0)
        drain(NCHUNK - 2, 0, ridx0, rows0, attr0, i0, g0, a0)
        pltpu.make_async_copy(row_hbm.at[s, 0], ridx0, i0).wait()
        drain(NCHUNK - 1, 0, ridx1, rows1, attr1, i1, g1, a1)
        pltpu.make_async_copy(row_hbm.at[s, 0], ridx1, i1).wait()
        plsc.subcore_barrier()

        for w in range(RPT // WB):
            pltpu.sync_copy(sx_acc.at[pl.ds(rbase + w * WB, WB)], stx)
            pltpu.sync_copy(stx, sx_out.at[c, pl.ds(rbase + w * WB, WB)])
            pltpu.sync_copy(se_acc.at[pl.ds(rbase + w * WB, WB)], ste)
            pltpu.sync_copy(ste, se_out.at[c, pl.ds(rbase + w * WB, WB)])

    return k(x2, row3, col3, attr3, ones_blk, z64, z16)


def _tc_epilogue(x, sx, se,
                 W1_0, b1_0, W1_1, b1_1, W1_2, b1_2,
                 W2_0, b2_0, W2_1, b2_1, W2_2, b2_2):
    def body(x_ref, sx_ref, se_ref,
             w10, b10, w11, b11, w12, b12,
             w20, b20, w21, b21, w22, b22, out_ref):
        f32 = jnp.float32
        dot = lambda a, b: jnp.dot(a, b, preferred_element_type=f32)
        t12 = dot(w11[...], w12[...])
        w1c = dot(w10[...], t12)
        b1c = dot(b10[...], t12) + dot(b11[...], w12[...]) + b12[...]
        t22 = dot(w21[...], w22[...])
        w2c = dot(w20[...], t22)
        b2c = dot(b20[...], t22) + dot(b21[...], w22[...]) + b22[...]
        p = w2c[:DN]
        bmat = w2c[DN:]
        q = dot(w1c[:DN], bmat)
        r = dot(w1c[DN:], bmat)
        svec = dot(b1c, bmat)
        out_ref[...] = (dot(x_ref[...], p)
                        + dot(sx_ref[0, :N], q[:DH])
                        + dot(sx_ref[1, :N], q[DH:])
                        + dot(se_ref[0, :N], r)
                        + se_ref[1, :N, 0:1] * svec + b2c)

    return pl.pallas_call(
        body,
        out_shape=jax.ShapeDtypeStruct((N, DN), jnp.float32),
    )(x, sx, se,
      W1_0, b1_0.reshape(1, -1), W1_1, b1_1.reshape(1, -1),
      W1_2, b1_2.reshape(1, -1),
      W2_0, b2_0.reshape(1, -1), W2_1, b2_1.reshape(1, -1),
      W2_2, b2_2.reshape(1, -1))


def kernel(x, edge_index, edge_attr, u, batch,
           W1_0, b1_0, W1_1, b1_1, W1_2, b1_2,
           W2_0, b2_0, W2_1, b2_1, W2_2, b2_2):
    pad = EPAD - E
    x2 = x.reshape(NC * N, DH)
    row3 = jnp.concatenate(
        [edge_index[0] * 2, jnp.zeros((pad,), jnp.int32)]
    ).reshape(NS, NCHUNK, CHUNK)
    col3 = jnp.concatenate(
        [edge_index[1], jnp.full((pad,), N, jnp.int32)]
    ).reshape(NS, NCHUNK, CHUNK)
    attr3 = jnp.concatenate(
        [edge_attr, jnp.zeros((pad, DE), jnp.float32)]
    ).reshape(NS * NCHUNK, CHUNK, DE)
    ones_blk = jnp.ones((CHUNK, DE), jnp.float32)
    z64 = jnp.zeros((WB, DH), jnp.float32)
    z16 = jnp.zeros((WB, DE), jnp.float32)
    sx, se = _sc_segment_sums(x2, row3, col3, attr3, ones_blk, z64, z16)
    return _tc_epilogue(x, sx, se,
                        W1_0, b1_0, W1_1, b1_1, W1_2, b1_2,
                        W2_0, b2_0, W2_1, b2_1, W2_2, b2_2)


# speedup vs baseline: 1.5054x; 1.3088x over previous
import functools

import jax
import jax.numpy as jnp
from jax import lax
from jax.experimental import pallas as pl
from jax.experimental.pallas import tpu as pltpu
from jax.experimental.pallas import tpu_sc as plsc

N = 10000
E = 320000
DN = 128
DE = 16
NC = 2
NS = 16
DH = DN // NC
CHUNK = 128
NCHUNK = 158
EPT = NCHUNK * CHUNK
EPAD = NS * EPT
NP = 10240
RPT = NP // NS
WB = 320
NV = CHUNK // 16


def _sc_segment_sums(x_flat, row4, col3, attr2, ones_blk, z64, z16):
    mesh = plsc.VectorSubcoreMesh(core_axis_name="c", subcore_axis_name="s")

    @functools.partial(
        pl.kernel,
        mesh=mesh,
        compiler_params=pltpu.CompilerParams(use_tc_tiling_on_sc=False),
        out_type=[
            jax.ShapeDtypeStruct((NC, NP, DH), jnp.float32),
            jax.ShapeDtypeStruct((NC, NP, DE), jnp.float32),
        ],
        scratch_types=[
            pltpu.VMEM((NCHUNK, CHUNK), jnp.int32),
            pltpu.VMEM((CHUNK,), jnp.int32),
            pltpu.VMEM((CHUNK,), jnp.int32),
            pltpu.VMEM((CHUNK, DH), jnp.float32),
            pltpu.VMEM((CHUNK, DH), jnp.float32),
            pltpu.VMEM((CHUNK, DE), jnp.float32),
            pltpu.VMEM((CHUNK, DE), jnp.float32),
            pltpu.VMEM((CHUNK, DE), jnp.float32),
            pltpu.VMEM((WB, DH), jnp.float32),
            pltpu.VMEM((WB, DE), jnp.float32),
            pltpu.VMEM_SHARED((NP, DH), jnp.float32),
            pltpu.VMEM_SHARED((NP, DE), jnp.float32),
            pltpu.SemaphoreType.DMA,
            pltpu.SemaphoreType.DMA,
            pltpu.SemaphoreType.DMA,
            pltpu.SemaphoreType.DMA,
            pltpu.SemaphoreType.DMA,
            pltpu.SemaphoreType.DMA,
        ],
    )
    def k(x_hbm, row_hbm, col_hbm, attr_hbm, ones_hbm, z64_hbm, z16_hbm,
          sx_out, se_out,
          cidx, ridx0, ridx1, rows0, rows1, attr0, attr1, onesb, stx, ste,
          sx_acc, se_acc, i0, i1, g0, g1, a0, a1):
        c = lax.axis_index("c")
        s = lax.axis_index("s")
        rbase = s * RPT
        on_c0 = c == 0

        pltpu.sync_copy(z64_hbm, stx)
        pltpu.sync_copy(z16_hbm, ste)
        for w in range(RPT // WB):
            pltpu.sync_copy(stx, sx_acc.at[pl.ds(rbase + w * WB, WB)])
            pltpu.sync_copy(ste, se_acc.at[pl.ds(rbase + w * WB, WB)])

        pltpu.sync_copy(col_hbm.at[s], cidx)
        pltpu.sync_copy(ones_hbm, onesb)
        plsc.subcore_barrier()

        def issue(j, ibuf, rbuf, abuf, isem, gsem, asem):
            pltpu.make_async_copy(row_hbm.at[c, s, 0], ibuf, isem).wait()
            pltpu.async_copy(x_hbm.at[ibuf], rbuf, gsem)

            @pl.when(jnp.logical_and(on_c0, s * EPT + j * CHUNK < E))
            def _():
                pltpu.async_copy(attr_hbm.at[pl.ds(s * EPT + j * CHUNK, CHUNK)],
                                 abuf, asem)

        def drain(j, jn, ibuf, rbuf, abuf, isem, gsem, asem):
            pltpu.make_async_copy(x_hbm.at[pl.ds(0, CHUNK)], rbuf, gsem).wait()
            pltpu.async_copy(row_hbm.at[c, s, jn], ibuf, isem)
            pltpu.sync_copy(rbuf, sx_acc.at[cidx.at[j]], add=True)

            @pl.when(jnp.logical_and(on_c0, s * EPT + j * CHUNK < E))
            def _():
                pltpu.make_async_copy(attr_hbm.at[pl.ds(0, CHUNK)], abuf,
                                      asem).wait()
                pltpu.sync_copy(abuf, se_acc.at[cidx.at[j]], add=True)

            @pl.when(jnp.logical_not(on_c0))
            def _():
                pltpu.sync_copy(onesb, se_acc.at[cidx.at[j]], add=True)

        pltpu.async_copy(row_hbm.at[c, s, 0], ridx0, i0)
        pltpu.async_copy(row_hbm.at[c, s, 1], ridx1, i1)
        issue(0, ridx0, rows0, attr0, i0, g0, a0)
        issue(1, ridx1, rows1, attr1, i1, g1, a1)

        def body(i, carry):
            j0 = 2 * i
            drain(j0, j0 + 2, ridx0, rows0, attr0, i0, g0, a0)
            issue(j0 + 2, ridx0, rows0, attr0, i0, g0, a0)
            drain(j0 + 1, j0 + 3, ridx1, rows1, attr1, i1, g1, a1)
            issue(j0 + 3, ridx1, rows1, attr1, i1, g1, a1)
            return carry

        lax.fori_loop(0, NCHUNK // 2 - 1, body, 0)
        drain(NCHUNK - 2, 0, ridx0, rows0, attr0, i0, g0, a0)
        pltpu.make_async_copy(row_hbm.at[c, s, 0], ridx0, i0).wait()
        drain(NCHUNK - 1, 0, ridx1, rows1, attr1, i1, g1, a1)
        pltpu.make_async_copy(row_hbm.at[c, s, 0], ridx1, i1).wait()
        plsc.subcore_barrier()

        for w in range(RPT // WB):
            pltpu.sync_copy(sx_acc.at[pl.ds(rbase + w * WB, WB)], stx)
            pltpu.sync_copy(stx, sx_out.at[c, pl.ds(rbase + w * WB, WB)])
            pltpu.sync_copy(se_acc.at[pl.ds(rbase + w * WB, WB)], ste)
            pltpu.sync_copy(ste, se_out.at[c, pl.ds(rbase + w * WB, WB)])

    return k(x_flat, row4, col3, attr2, ones_blk, z64, z16)


def _tc_epilogue(x, sx, se,
                 W1_0, b1_0, W1_1, b1_1, W1_2, b1_2,
                 W2_0, b2_0, W2_1, b2_1, W2_2, b2_2):
    def body(x_ref, sx_ref, se_ref,
             w10, b10, w11, b11, w12, b12,
             w20, b20, w21, b21, w22, b22, out_ref):
        f32 = jnp.float32
        dot = lambda a, b: jnp.dot(a, b, preferred_element_type=f32)
        t12 = dot(w11[...], w12[...])
        w1c = dot(w10[...], t12)
        b1c = dot(b10[...], t12) + dot(b11[...], w12[...]) + b12[...]
        t22 = dot(w21[...], w22[...])
        w2c = dot(w20[...], t22)
        b2c = dot(b20[...], t22) + dot(b21[...], w22[...]) + b22[...]
        p = w2c[:DN]
        bmat = w2c[DN:]
        q = dot(w1c[:DN], bmat)
        r = dot(w1c[DN:], bmat)
        svec = dot(b1c, bmat)
        out_ref[...] = (dot(x_ref[...], p)
                        + dot(sx_ref[0, :N], q[:DH])
                        + dot(sx_ref[1, :N], q[DH:])
                        + dot(se_ref[0, :N], r)
                        + se_ref[1, :N, 0:1] * svec + b2c)

    return pl.pallas_call(
        body,
        out_shape=jax.ShapeDtypeStruct((N, DN), jnp.float32),
    )(x, sx, se,
      W1_0, b1_0.reshape(1, -1), W1_1, b1_1.reshape(1, -1),
      W1_2, b1_2.reshape(1, -1),
      W2_0, b2_0.reshape(1, -1), W2_1, b2_1.reshape(1, -1),
      W2_2, b2_2.reshape(1, -1))


def kernel(x, edge_index, edge_attr, u, batch,
           W1_0, b1_0, W1_1, b1_1, W1_2, b1_2,
           W2_0, b2_0, W2_1, b2_1, W2_2, b2_2):
    pad = EPAD - E
    row = jnp.concatenate([edge_index[0], jnp.zeros((pad,), jnp.int32)])
    row4 = jnp.stack([row, row + N]).reshape(NC, NS, NCHUNK, CHUNK)
    col3 = jnp.concatenate(
        [edge_index[1], jnp.full((pad,), N, jnp.int32)]
    ).reshape(NS, NCHUNK, CHUNK)
    x_flat = jnp.concatenate([x[:, :DH], x[:, DH:]])
    ones_blk = jnp.ones((CHUNK, DE), jnp.float32)
    z64 = jnp.zeros((WB, DH), jnp.float32)
    z16 = jnp.zeros((WB, DE), jnp.float32)
    sx, se = _sc_segment_sums(x_flat, row4, col3, edge_attr, ones_blk, z64, z16)
    return _tc_epilogue(x, sx, se,
                        W1_0, b1_0, W1_1, b1_1, W1_2, b1_2,
                        W2_0, b2_0, W2_1, b2_1, W2_2, b2_2)


# speedup vs baseline: 1.5982x; 1.0617x over previous
import functools

import jax
import jax.numpy as jnp
from jax import lax
from jax.experimental import pallas as pl
from jax.experimental.pallas import tpu as pltpu
from jax.experimental.pallas import tpu_sc as plsc

N = 10000
E = 320000
DN = 128
DE = 16
NC = 2
NS = 16
DH = DN // NC
CHUNK = 128
NCHUNK = 158
EPT = NCHUNK * CHUNK
EPAD = NS * EPT
NP = 10240
RPT = NP // NS
WB = 320
NV = CHUNK // 16


def _sc_segment_sums(x_flat, row4, col3, attr2, ones_blk, z64, z16):
    mesh = plsc.VectorSubcoreMesh(core_axis_name="c", subcore_axis_name="s")

    @functools.partial(
        pl.kernel,
        mesh=mesh,
        compiler_params=pltpu.CompilerParams(use_tc_tiling_on_sc=False),
        out_type=[
            jax.ShapeDtypeStruct((NP, DN), jnp.float32),
            jax.ShapeDtypeStruct((NP, DN), jnp.float32),
        ],
        scratch_types=[
            pltpu.VMEM((NCHUNK, CHUNK), jnp.int32),
            pltpu.VMEM((CHUNK,), jnp.int32),
            pltpu.VMEM((CHUNK,), jnp.int32),
            pltpu.VMEM((CHUNK, DH), jnp.float32),
            pltpu.VMEM((CHUNK, DH), jnp.float32),
            pltpu.VMEM((CHUNK, DE), jnp.float32),
            pltpu.VMEM((CHUNK, DE), jnp.float32),
            pltpu.VMEM((CHUNK, DE), jnp.float32),
            pltpu.VMEM((WB, DH), jnp.float32),
            pltpu.VMEM((WB, DE), jnp.float32),
            pltpu.VMEM_SHARED((NP, DH), jnp.float32),
            pltpu.VMEM_SHARED((NP, DE), jnp.float32),
            pltpu.SemaphoreType.DMA,
            pltpu.SemaphoreType.DMA,
            pltpu.SemaphoreType.DMA,
            pltpu.SemaphoreType.DMA,
            pltpu.SemaphoreType.DMA,
            pltpu.SemaphoreType.DMA,
        ],
    )
    def k(x_hbm, row_hbm, col_hbm, attr_hbm, ones_hbm, z64_hbm, z16_hbm,
          sx_out, se_out,
          cidx, ridx0, ridx1, rows0, rows1, attr0, attr1, onesb, stx, ste,
          sx_acc, se_acc, i0, i1, g0, g1, a0, a1):
        c = lax.axis_index("c")
        s = lax.axis_index("s")
        rbase = s * RPT
        on_c0 = c == 0

        pltpu.sync_copy(z64_hbm, stx)
        pltpu.sync_copy(z16_hbm, ste)
        for w in range(RPT // WB):
            pltpu.sync_copy(stx, sx_acc.at[pl.ds(rbase + w * WB, WB)])
            pltpu.sync_copy(ste, se_acc.at[pl.ds(rbase + w * WB, WB)])

        pltpu.sync_copy(col_hbm.at[s], cidx)
        pltpu.sync_copy(ones_hbm, onesb)
        plsc.subcore_barrier()

        def issue(j, ibuf, rbuf, abuf, isem, gsem, asem):
            pltpu.make_async_copy(row_hbm.at[c, s, 0], ibuf, isem).wait()
            pltpu.async_copy(x_hbm.at[ibuf], rbuf, gsem)

            @pl.when(jnp.logical_and(on_c0, s * EPT + j * CHUNK < E))
            def _():
                pltpu.async_copy(attr_hbm.at[pl.ds(s * EPT + j * CHUNK, CHUNK)],
                                 abuf, asem)

        def drain(j, jn, ibuf, rbuf, abuf, isem, gsem, asem):
            pltpu.make_async_copy(x_hbm.at[pl.ds(0, CHUNK)], rbuf, gsem).wait()
            pltpu.async_copy(row_hbm.at[c, s, jn], ibuf, isem)
            pltpu.sync_copy(rbuf, sx_acc.at[cidx.at[j]], add=True)

            @pl.when(jnp.logical_and(on_c0, s * EPT + j * CHUNK < E))
            def _():
                pltpu.make_async_copy(attr_hbm.at[pl.ds(0, CHUNK)], abuf,
                                      asem).wait()
                pltpu.sync_copy(abuf, se_acc.at[cidx.at[j]], add=True)

            @pl.when(jnp.logical_not(on_c0))
            def _():
                pltpu.sync_copy(onesb, se_acc.at[cidx.at[j]], add=True)

        pltpu.async_copy(row_hbm.at[c, s, 0], ridx0, i0)
        pltpu.async_copy(row_hbm.at[c, s, 1], ridx1, i1)
        issue(0, ridx0, rows0, attr0, i0, g0, a0)
        issue(1, ridx1, rows1, attr1, i1, g1, a1)

        def body(i, carry):
            j0 = 2 * i
            drain(j0, j0 + 2, ridx0, rows0, attr0, i0, g0, a0)
            issue(j0 + 2, ridx0, rows0, attr0, i0, g0, a0)
            drain(j0 + 1, j0 + 3, ridx1, rows1, attr1, i1, g1, a1)
            issue(j0 + 3, ridx1, rows1, attr1, i1, g1, a1)
            return carry

        lax.fori_loop(0, NCHUNK // 2 - 1, body, 0)
        drain(NCHUNK - 2, 0, ridx0, rows0, attr0, i0, g0, a0)
        pltpu.make_async_copy(row_hbm.at[c, s, 0], ridx0, i0).wait()
        drain(NCHUNK - 1, 0, ridx1, rows1, attr1, i1, g1, a1)
        pltpu.make_async_copy(row_hbm.at[c, s, 0], ridx1, i1).wait()
        plsc.subcore_barrier()

        for w in range(RPT // WB):
            pltpu.sync_copy(sx_acc.at[pl.ds(rbase + w * WB, WB)], stx)
            pltpu.sync_copy(stx, sx_out.at[pl.ds(rbase + w * WB, WB),
                                           pl.ds(c * DH, DH)])
            pltpu.sync_copy(se_acc.at[pl.ds(rbase + w * WB, WB)], ste)
            pltpu.sync_copy(ste, se_out.at[pl.ds(rbase + w * WB, WB),
                                           pl.ds(c * DE, DE)])

    return k(x_flat, row4, col3, attr2, ones_blk, z64, z16)


def _tc_epilogue(x, sx, se,
                 W1_0, b1_0, W1_1, b1_1, W1_2, b1_2,
                 W2_0, b2_0, W2_1, b2_1, W2_2, b2_2):
    def body(x_ref, sx_ref, se_ref,
             w10, b10, w11, b11, w12, b12,
             w20, b20, w21, b21, w22, b22, out_ref):
        f32 = jnp.float32
        dot = lambda a, b: jnp.dot(a, b, preferred_element_type=f32)
        t12 = dot(w11[...], w12[...])
        w1c = dot(w10[...], t12)
        b1c = dot(b10[...], t12) + dot(b11[...], w12[...]) + b12[...]
        t22 = dot(w21[...], w22[...])
        w2c = dot(w20[...], t22)
        b2c = dot(b20[...], t22) + dot(b21[...], w22[...]) + b22[...]
        p = w2c[:DN]
        bmat = w2c[DN:]
        q = dot(w1c[:DN], bmat)
        r = dot(w1c[DN:], bmat)
        svec = dot(b1c, bmat)
        out_ref[...] = (dot(x_ref[...], p)
                        + dot(sx_ref[:N], q)
                        + dot(se_ref[:N, :DE], r)
                        + se_ref[:N, DE:DE + 1] * svec + b2c)

    return pl.pallas_call(
        body,
        out_shape=jax.ShapeDtypeStruct((N, DN), jnp.float32),
    )(x, sx, se,
      W1_0, b1_0.reshape(1, -1), W1_1, b1_1.reshape(1, -1),
      W1_2, b1_2.reshape(1, -1),
      W2_0, b2_0.reshape(1, -1), W2_1, b2_1.reshape(1, -1),
      W2_2, b2_2.reshape(1, -1))


def kernel(x, edge_index, edge_attr, u, batch,
           W1_0, b1_0, W1_1, b1_1, W1_2, b1_2,
           W2_0, b2_0, W2_1, b2_1, W2_2, b2_2):
    pad = EPAD - E
    row = jnp.concatenate([edge_index[0], jnp.zeros((pad,), jnp.int32)])
    row4 = jnp.stack([row, row + N]).reshape(NC, NS, NCHUNK, CHUNK)
    col3 = jnp.concatenate(
        [edge_index[1], jnp.full((pad,), N, jnp.int32)]
    ).reshape(NS, NCHUNK, CHUNK)
    x_flat = jnp.concatenate([x[:, :DH], x[:, DH:]])
    ones_blk = jnp.ones((CHUNK, DE), jnp.float32)
    z64 = jnp.zeros((WB, DH), jnp.float32)
    z16 = jnp.zeros((WB, DE), jnp.float32)
    sx, se = _sc_segment_sums(x_flat, row4, col3, edge_attr, ones_blk, z64, z16)
    return _tc_epilogue(x, sx, se,
                        W1_0, b1_0, W1_1, b1_1, W1_2, b1_2,
                        W2_0, b2_0, W2_1, b2_1, W2_2, b2_2)
